# baseline (speedup 1.0000x reference)
import functools

import jax
import jax.numpy as jnp
from jax import lax
from jax.experimental import pallas as pl
from jax.experimental.pallas import tpu as pltpu
from jax.experimental.pallas import tpu_sc as plsc

VOCAB = 100000
DIM = 128
CTX = 20
LATENT = 128

V_SC = 38400
V_TC = VOCAB - V_SC
RBLK = 15400
NBLK = V_TC // RBLK

NW = 32
RT = V_SC // NW
CH = 240
NCH = RT // CH
GRP = CH // 16


def _sc_gather(idx, table):
    mesh = plsc.VectorSubcoreMesh(core_axis_name="c", subcore_axis_name="s")

    @functools.partial(
        pl.kernel,
        mesh=mesh,
        out_type=jax.ShapeDtypeStruct((CTX, DIM), jnp.float32),
        scratch_types=[
            pltpu.VMEM((CTX,), jnp.int32),
            pltpu.VMEM((CTX, DIM), jnp.float32),
            pltpu.SemaphoreType.DMA,
        ],
    )
    def gather_k(idx_hbm, table_hbm, out_hbm, idx_v, rows_v, sem):
        wid = lax.axis_index("s") * 2 + lax.axis_index("c")

        @pl.when(wid == 0)
        def _():
            pltpu.sync_copy(idx_hbm, idx_v)
            pltpu.async_copy(table_hbm.at[idx_v], rows_v, sem).wait()
            pltpu.sync_copy(rows_v, out_hbm)

    return gather_k(idx, table)


def _perm16(vec, idx):
    return lax.gather(
        vec, idx[:, None],
        lax.GatherDimensionNumbers(offset_dims=(),
                                   collapsed_slice_dims=(0,),
                                   start_index_map=(0,)),
        slice_sizes=(1,),
        mode=lax.GatherScatterMode.PROMISE_IN_BOUNDS)


def _sc_dots(h8, W2):
    mesh = plsc.VectorSubcoreMesh(core_axis_name="c", subcore_axis_name="s")

    @functools.partial(
        pl.kernel,
        mesh=mesh,
        out_type=jax.ShapeDtypeStruct((V_SC,), jnp.float32),
        scratch_types=[
            pltpu.VMEM((8, 16), jnp.float32),
            pltpu.VMEM((CH, DIM), jnp.float32),
            pltpu.VMEM((CH, DIM), jnp.float32),
            pltpu.VMEM((RT,), jnp.float32),
            pltpu.SemaphoreType.DMA,
            pltpu.SemaphoreType.DMA,
        ],
    )
    def dots_k(h_hbm, w2_hbm, out_hbm, h_v, buf0, buf1, out_v, sem0, sem1):
        wid = lax.axis_index("s") * 2 + lax.axis_index("c")
        base = V_TC + wid * RT
        bufs = (buf0, buf1)
        sems = (sem0, sem1)

        pltpu.sync_copy(h_hbm, h_v)
        hv = [h_v[i] for i in range(8)]
        lanes = lax.iota(jnp.int32, 16)

        copies = [None, None]
        copies[0] = pltpu.async_copy(
            w2_hbm.at[pl.ds(base, CH)], buf0, sem0)

        for c in range(NCH):
            if c + 1 < NCH:
                copies[(c + 1) % 2] = pltpu.async_copy(
                    w2_hbm.at[pl.ds(base + (c + 1) * CH, CH)],
                    bufs[(c + 1) % 2], sems[(c + 1) % 2])
            copies[c % 2].wait()
            buf = bufs[c % 2]

            def group_body(g, carry, buf=buf, c=c):
                r0 = g * 16
                vecs = []
                for r in range(16):
                    acc = buf[r0 + r, pl.ds(0, 16)] * hv[0]
                    for i in range(1, 8):
                        acc = acc + buf[r0 + r, pl.ds(i * 16, 16)] * hv[i]
                    vecs.append(acc)
                for s in range(4):
                    stride = 1 << s
                    perm = lanes ^ stride
                    mask = (lanes & stride) == 0
                    nxt = []
                    for m in range(len(vecs) // 2):
                        u, v = vecs[2 * m], vecs[2 * m + 1]
                        nxt.append(jnp.where(mask, u + _perm16(u, perm),
                                             v + _perm16(v, perm)))
                    vecs = nxt
                out_v[pl.ds(c * CH + g * 16, 16)] = vecs[0]
                return carry

            lax.fori_loop(0, GRP, group_body, 0)

        pltpu.sync_copy(out_v, out_hbm.at[pl.ds(wid * RT, RT)])

    return dots_k(h8, W2)


def _h_kernel(x_ref, w1_ref, b1_ref, h_ref):
    h = lax.dot_general(x_ref[...], w1_ref[...],
                        (((1,), (1,)), ((), ())),
                        preferred_element_type=jnp.float32)
    h_ref[...] = jnp.maximum(h + b1_ref[...], 0.0)


def _tc_dots_kernel(h_ref, w2_ref, out_ref):
    dots = lax.dot_general(h_ref[...], w2_ref[...],
                           (((1,), (1,)), ((), ())),
                           preferred_element_type=jnp.float32)
    out_ref[0] = dots


def _final_kernel(dt_ref, ds_ref, bt_ref, bs_ref, ot_ref, os_ref):
    lt = dt_ref[...] + bt_ref[...]
    ls = ds_ref[...] + bs_ref[...]
    m = jnp.maximum(jnp.max(lt), jnp.max(ls))
    s = jnp.sum(jnp.exp(lt - m)) + jnp.sum(jnp.exp(ls - m))
    lse = m + jnp.log(s)
    ot_ref[...] = lt - lse
    os_ref[...] = ls - lse


def kernel(inputs, table, W1, b1, W2, b2):
    idx = inputs.astype(jnp.int32)
    embeds = _sc_gather(idx, table).reshape(1, CTX * DIM)

    h = pl.pallas_call(
        _h_kernel,
        out_shape=jax.ShapeDtypeStruct((1, LATENT), jnp.float32),
    )(embeds, W1, b1.reshape(1, LATENT))

    dots_sc = _sc_dots(h.reshape(8, 16), W2)

    dots_tc = pl.pallas_call(
        _tc_dots_kernel,
        grid=(NBLK,),
        in_specs=[
            pl.BlockSpec((1, LATENT), lambda b: (0, 0)),
            pl.BlockSpec((RBLK, DIM), lambda b: (b, 0)),
        ],
        out_specs=pl.BlockSpec((1, 1, RBLK), lambda b: (b, 0, 0)),
        out_shape=jax.ShapeDtypeStruct((NBLK, 1, RBLK), jnp.float32),
        compiler_params=pltpu.CompilerParams(
            dimension_semantics=("arbitrary",)),
    )(h, W2)

    out_tc, out_sc = pl.pallas_call(
        _final_kernel,
        out_shape=[
            jax.ShapeDtypeStruct((8, V_TC // 8), jnp.float32),
            jax.ShapeDtypeStruct((8, V_SC // 8), jnp.float32),
        ],
    )(dots_tc.reshape(8, V_TC // 8), dots_sc.reshape(8, V_SC // 8),
      b2[:V_TC].reshape(8, V_TC // 8), b2[V_TC:].reshape(8, V_SC // 8))

    return jnp.concatenate(
        [out_tc.reshape(1, V_TC), out_sc.reshape(1, V_SC)], axis=1)


# speedup vs baseline: 1.1454x; 1.1454x over previous
import functools

import jax
import jax.numpy as jnp
from jax import lax
from jax.experimental import pallas as pl
from jax.experimental.pallas import tpu as pltpu
from jax.experimental.pallas import tpu_sc as plsc

VOCAB = 100000
DIM = 128
CTX = 20
LATENT = 128

V_SC = 0
V_TC = VOCAB - V_SC
RBLK = 10000
NBLK = V_TC // RBLK

NW = 32
RT = V_SC // NW
CH = 240
NCH = RT // CH
GRP = CH // 16


def _sc_gather(idx, table):
    mesh = plsc.VectorSubcoreMesh(core_axis_name="c", subcore_axis_name="s")

    @functools.partial(
        pl.kernel,
        mesh=mesh,
        out_type=jax.ShapeDtypeStruct((CTX, DIM), jnp.float32),
        scratch_types=[
            pltpu.VMEM((CTX,), jnp.int32),
            pltpu.VMEM((CTX, DIM), jnp.float32),
            pltpu.SemaphoreType.DMA,
        ],
    )
    def gather_k(idx_hbm, table_hbm, out_hbm, idx_v, rows_v, sem):
        wid = lax.axis_index("s") * 2 + lax.axis_index("c")

        @pl.when(wid == 0)
        def _():
            pltpu.sync_copy(idx_hbm, idx_v)
            pltpu.async_copy(table_hbm.at[idx_v], rows_v, sem).wait()
            pltpu.sync_copy(rows_v, out_hbm)

    return gather_k(idx, table)


def _perm16(vec, idx):
    return lax.gather(
        vec, idx[:, None],
        lax.GatherDimensionNumbers(offset_dims=(),
                                   collapsed_slice_dims=(0,),
                                   start_index_map=(0,)),
        slice_sizes=(1,),
        mode=lax.GatherScatterMode.PROMISE_IN_BOUNDS)


def _sc_dots(h8, W2):
    mesh = plsc.VectorSubcoreMesh(core_axis_name="c", subcore_axis_name="s")

    @functools.partial(
        pl.kernel,
        mesh=mesh,
        out_type=jax.ShapeDtypeStruct((V_SC,), jnp.float32),
        scratch_types=[
            pltpu.VMEM((8, 16), jnp.float32),
            pltpu.VMEM((CH, DIM), jnp.float32),
            pltpu.VMEM((CH, DIM), jnp.float32),
            pltpu.VMEM((RT,), jnp.float32),
            pltpu.SemaphoreType.DMA,
            pltpu.SemaphoreType.DMA,
        ],
    )
    def dots_k(h_hbm, w2_hbm, out_hbm, h_v, buf0, buf1, out_v, sem0, sem1):
        wid = lax.axis_index("s") * 2 + lax.axis_index("c")
        base = V_TC + wid * RT
        bufs = (buf0, buf1)
        sems = (sem0, sem1)

        pltpu.sync_copy(h_hbm, h_v)
        hv = [h_v[i] for i in range(8)]
        lanes = lax.iota(jnp.int32, 16)

        copies = [None, None]
        copies[0] = pltpu.async_copy(
            w2_hbm.at[pl.ds(base, CH)], buf0, sem0)

        for c in range(NCH):
            if c + 1 < NCH:
                copies[(c + 1) % 2] = pltpu.async_copy(
                    w2_hbm.at[pl.ds(base + (c + 1) * CH, CH)],
                    bufs[(c + 1) % 2], sems[(c + 1) % 2])
            copies[c % 2].wait()
            buf = bufs[c % 2]

            def group_body(g, carry, buf=buf, c=c):
                r0 = g * 16
                vecs = []
                for r in range(16):
                    acc = buf[r0 + r, pl.ds(0, 16)] * hv[0]
                    for i in range(1, 8):
                        acc = acc + buf[r0 + r, pl.ds(i * 16, 16)] * hv[i]
                    vecs.append(acc)
                for s in range(4):
                    stride = 1 << s
                    perm = lanes ^ stride
                    mask = (lanes & stride) == 0
                    nxt = []
                    for m in range(len(vecs) // 2):
                        u, v = vecs[2 * m], vecs[2 * m + 1]
                        nxt.append(jnp.where(mask, u + _perm16(u, perm),
                                             v + _perm16(v, perm)))
                    vecs = nxt
                out_v[pl.ds(c * CH + g * 16, 16)] = vecs[0]
                return carry

            lax.fori_loop(0, GRP, group_body, 0)

        pltpu.sync_copy(out_v, out_hbm.at[pl.ds(wid * RT, RT)])

    return dots_k(h8, W2)


def _h_kernel(x_ref, w1_ref, b1_ref, h_ref):
    h = lax.dot_general(x_ref[...], w1_ref[...],
                        (((1,), (1,)), ((), ())),
                        preferred_element_type=jnp.float32)
    h_ref[...] = jnp.maximum(h + b1_ref[...], 0.0)


def _tc_dots_kernel(h_ref, w2_ref, out_ref):
    dots = lax.dot_general(h_ref[...], w2_ref[...],
                           (((1,), (1,)), ((), ())),
                           preferred_element_type=jnp.float32)
    out_ref[0] = dots


def _final_kernel(dt_ref, ds_ref, bt_ref, bs_ref, ot_ref, os_ref):
    lt = dt_ref[...] + bt_ref[...]
    ls = ds_ref[...] + bs_ref[...]
    m = jnp.maximum(jnp.max(lt), jnp.max(ls))
    s = jnp.sum(jnp.exp(lt - m)) + jnp.sum(jnp.exp(ls - m))
    lse = m + jnp.log(s)
    ot_ref[...] = lt - lse
    os_ref[...] = ls - lse


def _final_kernel_tc_only(dt_ref, bt_ref, ot_ref):
    lt = dt_ref[...] + bt_ref[...]
    m = jnp.max(lt)
    s = jnp.sum(jnp.exp(lt - m))
    lse = m + jnp.log(s)
    ot_ref[...] = lt - lse


def kernel(inputs, table, W1, b1, W2, b2):
    idx = inputs.astype(jnp.int32)
    embeds = _sc_gather(idx, table).reshape(1, CTX * DIM)

    h = pl.pallas_call(
        _h_kernel,
        out_shape=jax.ShapeDtypeStruct((1, LATENT), jnp.float32),
    )(embeds, W1, b1.reshape(1, LATENT))

    dots_sc = _sc_dots(h.reshape(8, 16), W2) if V_SC else None

    dots_tc = pl.pallas_call(
        _tc_dots_kernel,
        grid=(NBLK,),
        in_specs=[
            pl.BlockSpec((1, LATENT), lambda b: (0, 0)),
            pl.BlockSpec((RBLK, DIM), lambda b: (b, 0)),
        ],
        out_specs=pl.BlockSpec((1, 1, RBLK), lambda b: (b, 0, 0)),
        out_shape=jax.ShapeDtypeStruct((NBLK, 1, RBLK), jnp.float32),
        compiler_params=pltpu.CompilerParams(
            dimension_semantics=("arbitrary",)),
    )(h, W2)

    if not V_SC:
        out_tc = pl.pallas_call(
            _final_kernel_tc_only,
            out_shape=jax.ShapeDtypeStruct((8, V_TC // 8), jnp.float32),
        )(dots_tc.reshape(8, V_TC // 8), b2.reshape(8, V_TC // 8))
        return out_tc.reshape(1, VOCAB)

    out_tc, out_sc = pl.pallas_call(
        _final_kernel,
        out_shape=[
            jax.ShapeDtypeStruct((8, V_TC // 8), jnp.float32),
            jax.ShapeDtypeStruct((8, V_SC // 8), jnp.float32),
        ],
    )(dots_tc.reshape(8, V_TC // 8), dots_sc.reshape(8, V_SC // 8),
      b2[:V_TC].reshape(8, V_TC // 8), b2[V_TC:].reshape(8, V_SC // 8))

    return jnp.concatenate(
        [out_tc.reshape(1, V_TC), out_sc.reshape(1, V_SC)], axis=1)


# speedup vs baseline: 1.1615x; 1.0141x over previous
import functools

import jax
import jax.numpy as jnp
from jax import lax
from jax.experimental import pallas as pl
from jax.experimental.pallas import tpu as pltpu
from jax.experimental.pallas import tpu_sc as plsc

VOCAB = 100000
DIM = 128
CTX = 20
LATENT = 128

RBLK = 10000
NBLK = VOCAB // RBLK


def _sc_gather(idx, table):
    mesh = plsc.VectorSubcoreMesh(core_axis_name="c", subcore_axis_name="s")

    @functools.partial(
        pl.kernel,
        mesh=mesh,
        out_type=jax.ShapeDtypeStruct((CTX, DIM), jnp.float32),
        scratch_types=[
            pltpu.VMEM((CTX,), jnp.int32),
            pltpu.VMEM((CTX, DIM), jnp.float32),
            pltpu.SemaphoreType.DMA,
        ],
    )
    def gather_k(idx_hbm, table_hbm, out_hbm, idx_v, rows_v, sem):
        wid = lax.axis_index("s") * 2 + lax.axis_index("c")

        @pl.when(wid == 0)
        def _():
            pltpu.sync_copy(idx_hbm, idx_v)
            pltpu.async_copy(table_hbm.at[idx_v], rows_v, sem).wait()
            pltpu.sync_copy(rows_v, out_hbm)

    return gather_k(idx, table)


def _logits_kernel(x_ref, w1_ref, b1_ref, w2_ref, b2_ref,
                   logits_ref, lse_ref, h_scr, acc_scr):
    b = pl.program_id(0)

    @pl.when(b == 0)
    def _():
        h = lax.dot_general(x_ref[...], w1_ref[...],
                            (((1,), (1,)), ((), ())),
                            preferred_element_type=jnp.float32)
        h_scr[0:1, :] = jnp.maximum(h + b1_ref[...], 0.0)
        acc_scr[0] = -jnp.inf
        acc_scr[1] = 0.0

    h = h_scr[0:1, :]
    w2 = w2_ref[0]
    logits = lax.dot_general(h, w2, (((1,), (1,)), ((), ())),
                             preferred_element_type=jnp.float32)
    logits = logits + b2_ref[0]
    logits_ref[0] = logits

    m_old = acc_scr[0]
    m_new = jnp.maximum(m_old, jnp.max(logits))
    s_new = (acc_scr[1] * jnp.exp(m_old - m_new)
             + jnp.sum(jnp.exp(logits - m_new)))
    acc_scr[0] = m_new
    acc_scr[1] = s_new

    @pl.when(b == NBLK - 1)
    def _():
        lse_ref[0, 0] = m_new + jnp.log(s_new)


def _norm_kernel(logits_ref, lse_ref, out_ref):
    out_ref[...] = logits_ref[...] - lse_ref[0, 0]


def kernel(inputs, table, W1, b1, W2, b2):
    idx = inputs.astype(jnp.int32)
    embeds = _sc_gather(idx, table).reshape(1, CTX * DIM)

    w2_blocks = W2.reshape(NBLK, RBLK, DIM)
    b2_blocks = b2.reshape(NBLK, 1, RBLK)

    logits, lse = pl.pallas_call(
        _logits_kernel,
        grid=(NBLK,),
        in_specs=[
            pl.BlockSpec((1, CTX * DIM), lambda b: (0, 0)),
            pl.BlockSpec((LATENT, CTX * DIM), lambda b: (0, 0)),
            pl.BlockSpec((1, LATENT), lambda b: (0, 0)),
            pl.BlockSpec((1, RBLK, DIM), lambda b: (b, 0, 0)),
            pl.BlockSpec((1, 1, RBLK), lambda b: (b, 0, 0)),
        ],
        out_specs=[
            pl.BlockSpec((1, 1, RBLK), lambda b: (b, 0, 0)),
            pl.BlockSpec(memory_space=pltpu.SMEM),
        ],
        out_shape=[
            jax.ShapeDtypeStruct((NBLK, 1, RBLK), jnp.float32),
            jax.ShapeDtypeStruct((1, 1), jnp.float32),
        ],
        scratch_shapes=[
            pltpu.VMEM((8, LATENT), jnp.float32),
            pltpu.SMEM((2,), jnp.float32),
        ],
        compiler_params=pltpu.CompilerParams(
            dimension_semantics=("arbitrary",)),
    )(embeds, W1, b1.reshape(1, LATENT), w2_blocks, b2_blocks)

    log_probs = pl.pallas_call(
        _norm_kernel,
        in_specs=[
            pl.BlockSpec((NBLK, 1, RBLK), lambda: (0, 0, 0)),
            pl.BlockSpec(memory_space=pltpu.SMEM),
        ],
        out_specs=pl.BlockSpec((NBLK, 1, RBLK), lambda: (0, 0, 0)),
        out_shape=jax.ShapeDtypeStruct((NBLK, 1, RBLK), jnp.float32),
    )(logits, lse)

    return log_probs.reshape(1, VOCAB)
